# baseline (speedup 1.0000x reference)
import functools

import jax
import jax.numpy as jnp
from jax import lax
from jax.experimental import pallas as pl
from jax.experimental.pallas import tpu as pltpu
from jax.experimental.pallas import tpu_sc as plsc

_N = 10000
_E = 320000
_R = 16
_D = 128
_H = 128
_L = 16

_NC = 2
_NS = 16
_NW = _NC * _NS
_EPT = _E // _NW
_CH = 80
_NCHUNK = _EPT // _CH
_NR = _N * _R
_NPAD = 10240
_NPT = _NPAD // _NS

_BN = 400
_NBLK = _N // _BN


def _sc_mesh():
    return plsc.VectorSubcoreMesh(core_axis_name="c", subcore_axis_name="s")


def _fill_rows_zero(rows, nrows, ncol):
    z = jnp.zeros((16,), jnp.float32)

    def b(i, c):
        for q in range(ncol // 16):
            rows[i, pl.ds(q * 16, 16)] = z
        return c

    lax.fori_loop(0, nrows, b, 0)


def _spmem_rows_zero_and_sync(rows, agg_s, sid, nrow_chunk, ncol):
    _fill_rows_zero(rows, nrow_chunk, ncol)
    base = sid * _NPT
    for kk in range(_NPT // nrow_chunk):
        pltpu.sync_copy(rows, agg_s.at[pl.ds(base + kk * nrow_chunk,
                                             nrow_chunk)])


def _spmem_rows_out(rows, agg_s, agg_out, cid, sid, nrow_chunk):
    base = sid * _NPT
    for kk in range(_NPT // nrow_chunk):
        sl = pl.ds(base + kk * nrow_chunk, nrow_chunk)
        pltpu.sync_copy(agg_s.at[sl], rows)
        pltpu.sync_copy(rows, agg_out.at[cid, sl])


def _sc_counts(src, dst, rel):
    @functools.partial(
        pl.kernel,
        out_type=(
            jax.ShapeDtypeStruct((_NC * _NR,), jnp.float32),
            jax.ShapeDtypeStruct((_NW, _NCHUNK, _CH), jnp.int32),
            jax.ShapeDtypeStruct((_NW, _NCHUNK, _CH), jnp.int32),
        ),
        mesh=_sc_mesh(),
        scratch_types=(
            pltpu.VMEM((_EPT,), jnp.int32),
            pltpu.VMEM((_EPT,), jnp.int32),
            pltpu.VMEM((_EPT,), jnp.int32),
            pltpu.VMEM((_NCHUNK, _CH), jnp.int32),
            pltpu.VMEM((_NCHUNK, _CH), jnp.int32),
            pltpu.VMEM((_CH,), jnp.float32),
            pltpu.VMEM((_EPT,), jnp.float32),
            pltpu.VMEM_SHARED((_NR,), jnp.float32),
            pltpu.SemaphoreType.DMA,
        ),
    )
    def k(src_h, dst_h, rel_h, cnt_out, ks_out, kd_out,
          srcv, dstv, relv, ksv2, kdv2, onesv, zv, cnt_s, sems):
        cid = lax.axis_index("c")
        sid = lax.axis_index("s")
        wid = cid * _NS + sid
        base = wid * _EPT
        seg = _NR // _NS

        pltpu.sync_copy(src_h.at[pl.ds(base, _EPT)], srcv)
        pltpu.sync_copy(dst_h.at[pl.ds(base, _EPT)], dstv)
        pltpu.sync_copy(rel_h.at[pl.ds(base, _EPT)], relv)

        z = jnp.zeros((16,), jnp.float32)

        def zf(i, c):
            zv[pl.ds(i * 16, 16)] = z
            return c

        lax.fori_loop(0, _EPT // 16, zf, 0)
        pltpu.sync_copy(zv, cnt_s.at[pl.ds(sid * seg, seg)])
        for i in range(_CH // 16):
            onesv[pl.ds(i * 16, 16)] = jnp.ones((16,), jnp.float32)

        def keys(j, carry):
            for q in range(_CH // 16):
                sl = pl.ds(q * 16, 16)
                fl = pl.ds(j * _CH + q * 16, 16)
                rv = relv[fl]
                ksv2[j, sl] = srcv[fl] * _R + rv
                kdv2[j, sl] = dstv[fl] * _R + rv
            return carry

        lax.fori_loop(0, _NCHUNK, keys, 0)
        pltpu.sync_copy(ksv2, ks_out.at[wid])
        pltpu.sync_copy(kdv2, kd_out.at[wid])
        plsc.subcore_barrier()

        def chunk(j, carry):
            pltpu.async_copy(onesv, cnt_s.at[kdv2.at[j]], sems, add=True)

            @pl.when(j >= 1)
            def _():
                pltpu.make_async_copy(cnt_out.at[pl.ds(0, _CH)],
                                      onesv, sems).wait()

            return carry

        lax.fori_loop(0, _NCHUNK, chunk, 0)
        pltpu.make_async_copy(cnt_out.at[pl.ds(0, _CH)], onesv, sems).wait()
        plsc.subcore_barrier()
        sl = pl.ds(sid * seg, seg)
        pltpu.sync_copy(cnt_s.at[sl], zv)
        pltpu.sync_copy(zv, cnt_out.at[pl.ds(cid * _NR + sid * seg, seg)])

    return k(src, dst, rel)


def _sc_invn(kd3, invc):
    @functools.partial(
        pl.kernel,
        out_type=jax.ShapeDtypeStruct((_NW, _NCHUNK, _CH), jnp.float32),
        mesh=_sc_mesh(),
        scratch_types=(
            pltpu.VMEM((_NCHUNK, _CH), jnp.int32),
            pltpu.VMEM((2, _CH), jnp.float32),
            pltpu.SemaphoreType.DMA,
            pltpu.SemaphoreType.DMA,
        ),
    )
    def k(kd_h, invc_h, invn_out, kdv2, invs, semi, semo):
        cid = lax.axis_index("c")
        sid = lax.axis_index("s")
        wid = cid * _NS + sid

        pltpu.sync_copy(kd_h.at[wid], kdv2)
        pltpu.async_copy(invc_h.at[kdv2.at[0]], invs.at[0], semi)

        def chunk(j, carry):
            jm2 = lax.rem(j, 2)

            @pl.when(j >= 1)
            def _():
                pltpu.make_async_copy(invn_out.at[0, 0], invs.at[0],
                                      semo).wait()

            @pl.when(j + 1 < _NCHUNK)
            def _():
                pltpu.async_copy(invc_h.at[kdv2.at[j + 1]],
                                 invs.at[1 - jm2], semi)

            pltpu.make_async_copy(invc_h.at[pl.ds(0, _CH)], invs.at[0],
                                  semi).wait()
            pltpu.async_copy(invs.at[jm2], invn_out.at[wid, j], semo)
            return carry

        lax.fori_loop(0, _NCHUNK, chunk, 0)
        pltpu.make_async_copy(invn_out.at[0, 0], invs.at[0], semo).wait()

    return k(kd3, invc)


def _sc_edge1(xw1, ks3, dst3, invn3):
    @functools.partial(
        pl.kernel,
        out_type=jax.ShapeDtypeStruct((_NC, _NPAD, _H), jnp.float32),
        mesh=_sc_mesh(),
        scratch_types=(
            pltpu.VMEM((3, _CH), jnp.int32),
            pltpu.VMEM((_NCHUNK, _CH), jnp.int32),
            pltpu.VMEM((3, _CH), jnp.float32),
            pltpu.VMEM((3, _CH, _H), jnp.float32),
            pltpu.VMEM_SHARED((_NPAD, _H), jnp.float32),
            pltpu.SemaphoreType.DMA,
            pltpu.SemaphoreType.DMA,
            pltpu.SemaphoreType.DMA,
            pltpu.SemaphoreType.DMA,
        ),
    )
    def k(xw_h, ks_h, dst_h, invn_h, agg_out,
          ksst, dstv2, invs, rows3, agg_s, semr, semn, semk, semsc):
        cid = lax.axis_index("c")
        sid = lax.axis_index("s")
        wid = cid * _NS + sid

        pltpu.sync_copy(dst_h.at[wid], dstv2)
        for p in range(3):
            pltpu.sync_copy(ks_h.at[wid, p], ksst.at[p])
        _spmem_rows_zero_and_sync(rows3.at[0], agg_s, sid, _CH, _H)
        plsc.subcore_barrier()

        def fire(j, buf):
            pltpu.async_copy(xw_h.at[ksst.at[buf]], rows3.at[buf], semr)
            pltpu.async_copy(invn_h.at[wid, j], invs.at[buf], semn)

        fire(0, 0)

        def chunk(j, carry):
            jm3 = lax.rem(j, 3)
            jn3 = lax.rem(j + 1, 3)

            @pl.when(j >= 1)
            def _():
                pltpu.make_async_copy(xw_h.at[pl.ds(0, _CH)],
                                      rows3.at[0], semsc).wait()

            @pl.when(jnp.logical_and(j >= 2, j + 1 < _NCHUNK))
            def _():
                pltpu.make_async_copy(ks_h.at[0, 0], ksst.at[0], semk).wait()

            @pl.when(j + 1 < _NCHUNK)
            def _():
                fire(j + 1, jn3)

            pltpu.make_async_copy(xw_h.at[pl.ds(0, _CH)],
                                  rows3.at[0], semr).wait()
            pltpu.make_async_copy(invn_h.at[0, 0], invs.at[0], semn).wait()

            @pl.when(j + 3 < _NCHUNK)
            def _():
                pltpu.async_copy(ks_h.at[wid, j + 3], ksst.at[jm3], semk)

            def blk(bb, c2):
                iv = invs[jm3, pl.ds(bb * 16, 16)]
                for ii in range(16):
                    i = bb * 16 + ii
                    inv = jnp.full((16,), iv[ii], jnp.float32)
                    for q in range(_H // 16):
                        sl = pl.ds(q * 16, 16)
                        rows3[jm3, i, sl] = rows3[jm3, i, sl] * inv
                return c2

            lax.fori_loop(0, _CH // 16, blk, 0)
            pltpu.async_copy(rows3.at[jm3], agg_s.at[dstv2.at[j]], semsc,
                             add=True)
            return carry

        lax.fori_loop(0, _NCHUNK, chunk, 0)
        pltpu.make_async_copy(xw_h.at[pl.ds(0, _CH)], rows3.at[0],
                              semsc).wait()
        plsc.subcore_barrier()
        _spmem_rows_out(rows3.at[0], agg_s, agg_out, cid, sid, _CH)

    return k(xw1, ks3, dst3, invn3)


def _sc_edge2(xw2, ks, dst3, invn3):
    @functools.partial(
        pl.kernel,
        out_type=jax.ShapeDtypeStruct((_NC, _NPAD, _L), jnp.float32),
        mesh=_sc_mesh(),
        compiler_params=pltpu.CompilerParams(use_tc_tiling_on_sc=False),
        scratch_types=(
            pltpu.VMEM((_EPT,), jnp.int32),
            pltpu.VMEM((_NCHUNK, _CH), jnp.int32),
            pltpu.VMEM((_NCHUNK, _CH), jnp.float32),
            pltpu.VMEM((3, _CH, _L), jnp.float32),
            pltpu.VMEM_SHARED((_NPAD, _L), jnp.float32),
            pltpu.SemaphoreType.DMA,
            pltpu.SemaphoreType.DMA,
        ),
    )
    def k(xw_h, ks_h, dst_h, invn_h, agg_out,
          ksv, dstv2, invnv2, rows3, agg_s, semr, semsc):
        cid = lax.axis_index("c")
        sid = lax.axis_index("s")
        wid = cid * _NS + sid
        base = wid * _EPT

        pltpu.sync_copy(ks_h.at[pl.ds(base, _EPT)], ksv)
        pltpu.sync_copy(dst_h.at[wid], dstv2)
        pltpu.sync_copy(invn_h.at[wid], invnv2)
        _spmem_rows_zero_and_sync(rows3.at[0], agg_s, sid, _CH, _L)
        plsc.subcore_barrier()

        def fire(j, buf):
            pltpu.async_copy(xw_h.at[ksv.at[pl.ds(j * _CH, _CH)]],
                             rows3.at[buf], semr)

        fire(0, 0)

        def chunk(j, carry):
            jm3 = lax.rem(j, 3)
            jn3 = lax.rem(j + 1, 3)

            @pl.when(j >= 1)
            def _():
                pltpu.make_async_copy(xw_h.at[pl.ds(0, _CH)],
                                      rows3.at[0], semsc).wait()

            @pl.when(j + 1 < _NCHUNK)
            def _():
                fire(j + 1, jn3)

            pltpu.make_async_copy(xw_h.at[pl.ds(0, _CH)],
                                  rows3.at[0], semr).wait()

            def blk(bb, c2):
                iv = invnv2[j, pl.ds(bb * 16, 16)]
                for ii in range(16):
                    i = bb * 16 + ii
                    inv = jnp.full((16,), iv[ii], jnp.float32)
                    rows3[jm3, i, pl.ds(0, _L)] = (
                        rows3[jm3, i, pl.ds(0, _L)] * inv)
                return c2

            lax.fori_loop(0, _CH // 16, blk, 0)
            pltpu.async_copy(rows3.at[jm3], agg_s.at[dstv2.at[j]], semsc,
                             add=True)
            return carry

        lax.fori_loop(0, _NCHUNK, chunk, 0)
        pltpu.make_async_copy(xw_h.at[pl.ds(0, _CH)], rows3.at[0],
                              semsc).wait()
        plsc.subcore_barrier()
        _spmem_rows_out(rows3.at[0], agg_s, agg_out, cid, sid, _CH)

    return k(xw2, ks, dst3, invn3)


def _tc_pre(emb, wc, bc8, w1c, root1, b18):
    def body(emb_ref, wc_ref, bc_ref, w1c_ref, r1_ref, b1_ref,
             x1_ref, xw1_ref, rt1_ref):
        bc = bc_ref[...][0:1, :]
        b1 = b1_ref[...][0:1, :]
        x = jnp.dot(emb_ref[...], wc_ref[...],
                    preferred_element_type=jnp.float32) + bc
        x1_ref[...] = x
        xw1_ref[...] = jnp.dot(x, w1c_ref[...],
                               preferred_element_type=jnp.float32)
        rt1_ref[...] = jnp.dot(x, r1_ref[...],
                               preferred_element_type=jnp.float32) + b1

    return pl.pallas_call(
        body,
        grid=(_NBLK,),
        in_specs=[
            pl.BlockSpec((_BN, _D), lambda i: (i, 0)),
            pl.BlockSpec((_D, _D), lambda i: (0, 0)),
            pl.BlockSpec((8, _D), lambda i: (0, 0)),
            pl.BlockSpec((_D, _R * _H), lambda i: (0, 0)),
            pl.BlockSpec((_D, _H), lambda i: (0, 0)),
            pl.BlockSpec((8, _H), lambda i: (0, 0)),
        ],
        out_specs=[
            pl.BlockSpec((_BN, _D), lambda i: (i, 0)),
            pl.BlockSpec((_BN, _R * _H), lambda i: (i, 0)),
            pl.BlockSpec((_BN, _H), lambda i: (i, 0)),
        ],
        out_shape=[
            jax.ShapeDtypeStruct((_N, _D), jnp.float32),
            jax.ShapeDtypeStruct((_N, _R * _H), jnp.float32),
            jax.ShapeDtypeStruct((_N, _H), jnp.float32),
        ],
    )(emb, wc, bc8, w1c, root1, b18)


def _tc_invc(cnt2):
    def body(cnt_ref, invc_ref):
        c = cnt_ref[0] + cnt_ref[1]
        invc_ref[...] = 1.0 / jnp.maximum(c, 1.0)

    return pl.pallas_call(
        body,
        out_shape=jax.ShapeDtypeStruct((_NR // 128, 128), jnp.float32),
    )(cnt2)


def _tc_mid(agg1, rt1, w2c, root2, b28):
    def body(agg_ref, rt1_ref, w2c_ref, r2_ref, b2_ref, xw2_ref, rt2_ref):
        b2 = b2_ref[...][0:1, :]
        x2 = jax.nn.relu(agg_ref[0] + agg_ref[1] + rt1_ref[...])
        xw2_ref[...] = jnp.dot(x2, w2c_ref[...],
                               preferred_element_type=jnp.float32)
        rt2_ref[...] = jnp.dot(x2, r2_ref[...],
                               preferred_element_type=jnp.float32) + b2

    return pl.pallas_call(
        body,
        grid=(_NBLK,),
        in_specs=[
            pl.BlockSpec((2, _BN, _H), lambda i: (0, i, 0)),
            pl.BlockSpec((_BN, _H), lambda i: (i, 0)),
            pl.BlockSpec((_H, _R * _L), lambda i: (0, 0)),
            pl.BlockSpec((_H, _L), lambda i: (0, 0)),
            pl.BlockSpec((8, _L), lambda i: (0, 0)),
        ],
        out_specs=[
            pl.BlockSpec((_BN, _R * _L), lambda i: (i, 0)),
            pl.BlockSpec((_BN, _L), lambda i: (i, 0)),
        ],
        out_shape=[
            jax.ShapeDtypeStruct((_N, _R * _L), jnp.float32),
            jax.ShapeDtypeStruct((_N, _L), jnp.float32),
        ],
    )(agg1, rt1, w2c, root2, b28)


def _tc_final(agg2, rt2):
    def body(agg_ref, rt2_ref, out_ref):
        y = agg_ref[0] + agg_ref[1] + rt2_ref[...]
        m = jnp.max(y, axis=-1, keepdims=True)
        e = jnp.exp(y - m)
        out_ref[...] = e / jnp.sum(e, axis=-1, keepdims=True)

    return pl.pallas_call(
        body,
        grid=(_NBLK,),
        in_specs=[
            pl.BlockSpec((2, _BN, _L), lambda i: (0, i, 0)),
            pl.BlockSpec((_BN, _L), lambda i: (i, 0)),
        ],
        out_specs=pl.BlockSpec((_BN, _L), lambda i: (i, 0)),
        out_shape=jax.ShapeDtypeStruct((_N, _L), jnp.float32),
    )(agg2, rt2)


def kernel(embedding, edge_index, edge_type, in_proj_w, in_proj_b,
           out_proj_w, out_proj_b, w1, root1, b1, w2, root2, b2):
    emb = embedding[0]
    wv = in_proj_w[2 * _D:3 * _D]
    bv = in_proj_b[2 * _D:3 * _D]
    wc = wv.T @ out_proj_w.T
    bc = bv @ out_proj_w.T + out_proj_b
    bc8 = jnp.broadcast_to(bc[None, :], (8, _D))
    b18 = jnp.broadcast_to(b1[None, :], (8, _H))
    b28 = jnp.broadcast_to(b2[None, :], (8, _L))
    w1c = w1.transpose(1, 0, 2).reshape(_D, _R * _H)
    w2c = w2.transpose(1, 0, 2).reshape(_H, _R * _L)

    src = edge_index[0]
    dst = edge_index[1]
    rel = edge_type
    dst3 = dst.reshape(_NW, _NCHUNK, _CH)

    cnt2, ks3, kd3 = _sc_counts(src, dst, rel)
    ks = ks3.reshape(_E)
    invc = _tc_invc(cnt2.reshape(2, _NR // 128, 128))
    invn3 = _sc_invn(kd3, invc.reshape(_NR))
    _, xw1, rt1 = _tc_pre(emb, wc, bc8, w1c, root1, b18)
    agg1 = _sc_edge1(xw1.reshape(_NR, _H), ks3, dst3, invn3)
    xw2, rt2 = _tc_mid(agg1, rt1, w2c, root2, b28)
    agg2 = _sc_edge2(xw2.reshape(_NR, _L), ks, dst3, invn3)
    return _tc_final(agg2, rt2)


# speedup vs baseline: 1.0882x; 1.0882x over previous
import functools

import jax
import jax.numpy as jnp
from jax import lax
from jax.experimental import pallas as pl
from jax.experimental.pallas import tpu as pltpu
from jax.experimental.pallas import tpu_sc as plsc

_N = 10000
_E = 320000
_R = 16
_D = 128
_H = 128
_L = 16

_NC = 2
_NS = 16
_NW = _NC * _NS
_EPT = _E // _NW
_CH = 80
_NCHUNK = _EPT // _CH
_NR = _N * _R
_NPAD = 10240
_NPT = _NPAD // _NS

_BN = 400
_NBLK = _N // _BN


def _sc_mesh():
    return plsc.VectorSubcoreMesh(core_axis_name="c", subcore_axis_name="s")


def _fill_rows_zero(rows, nrows, ncol):
    z = jnp.zeros((16,), jnp.float32)

    def b(i, c):
        for q in range(ncol // 16):
            rows[i, pl.ds(q * 16, 16)] = z
        return c

    lax.fori_loop(0, nrows, b, 0)


def _spmem_rows_zero_and_sync(rows, agg_s, sid, nrow_chunk, ncol):
    _fill_rows_zero(rows, nrow_chunk, ncol)
    base = sid * _NPT
    for kk in range(_NPT // nrow_chunk):
        pltpu.sync_copy(rows, agg_s.at[pl.ds(base + kk * nrow_chunk,
                                             nrow_chunk)])


def _spmem_rows_out(rows, agg_s, agg_out, cid, sid, nrow_chunk):
    base = sid * _NPT
    for kk in range(_NPT // nrow_chunk):
        sl = pl.ds(base + kk * nrow_chunk, nrow_chunk)
        pltpu.sync_copy(agg_s.at[sl], rows)
        pltpu.sync_copy(rows, agg_out.at[cid, sl])


def _sc_counts(src, dst, rel):
    @functools.partial(
        pl.kernel,
        out_type=(
            jax.ShapeDtypeStruct((_NC * _NR,), jnp.float32),
            jax.ShapeDtypeStruct((_NW, _NCHUNK, _CH), jnp.int32),
            jax.ShapeDtypeStruct((_NW, _NCHUNK, _CH), jnp.int32),
        ),
        mesh=_sc_mesh(),
        scratch_types=(
            pltpu.VMEM((_EPT,), jnp.int32),
            pltpu.VMEM((_EPT,), jnp.int32),
            pltpu.VMEM((_EPT,), jnp.int32),
            pltpu.VMEM((_NCHUNK, _CH), jnp.int32),
            pltpu.VMEM((_NCHUNK, _CH), jnp.int32),
            pltpu.VMEM((_CH,), jnp.float32),
            pltpu.VMEM((_EPT,), jnp.float32),
            pltpu.VMEM_SHARED((_NR,), jnp.float32),
            pltpu.SemaphoreType.DMA,
        ),
    )
    def k(src_h, dst_h, rel_h, cnt_out, ks_out, kd_out,
          srcv, dstv, relv, ksv2, kdv2, onesv, zv, cnt_s, sems):
        cid = lax.axis_index("c")
        sid = lax.axis_index("s")
        wid = cid * _NS + sid
        base = wid * _EPT
        seg = _NR // _NS

        pltpu.sync_copy(src_h.at[pl.ds(base, _EPT)], srcv)
        pltpu.sync_copy(dst_h.at[pl.ds(base, _EPT)], dstv)
        pltpu.sync_copy(rel_h.at[pl.ds(base, _EPT)], relv)

        z = jnp.zeros((16,), jnp.float32)

        def zf(i, c):
            zv[pl.ds(i * 16, 16)] = z
            return c

        lax.fori_loop(0, _EPT // 16, zf, 0)
        pltpu.sync_copy(zv, cnt_s.at[pl.ds(sid * seg, seg)])
        for i in range(_CH // 16):
            onesv[pl.ds(i * 16, 16)] = jnp.ones((16,), jnp.float32)

        def keys(j, carry):
            for q in range(_CH // 16):
                sl = pl.ds(q * 16, 16)
                fl = pl.ds(j * _CH + q * 16, 16)
                rv = relv[fl]
                ksv2[j, sl] = srcv[fl] * _R + rv
                kdv2[j, sl] = dstv[fl] * _R + rv
            return carry

        lax.fori_loop(0, _NCHUNK, keys, 0)
        pltpu.sync_copy(ksv2, ks_out.at[wid])
        pltpu.sync_copy(kdv2, kd_out.at[wid])
        plsc.subcore_barrier()

        def chunk(j, carry):
            pltpu.async_copy(onesv, cnt_s.at[kdv2.at[j]], sems, add=True)

            @pl.when(j >= 1)
            def _():
                pltpu.make_async_copy(cnt_out.at[pl.ds(0, _CH)],
                                      onesv, sems).wait()

            return carry

        lax.fori_loop(0, _NCHUNK, chunk, 0)
        pltpu.make_async_copy(cnt_out.at[pl.ds(0, _CH)], onesv, sems).wait()
        plsc.subcore_barrier()
        sl = pl.ds(sid * seg, seg)
        pltpu.sync_copy(cnt_s.at[sl], zv)
        pltpu.sync_copy(zv, cnt_out.at[pl.ds(cid * _NR + sid * seg, seg)])

    return k(src, dst, rel)


def _sc_invn(kd3, invc):
    @functools.partial(
        pl.kernel,
        out_type=jax.ShapeDtypeStruct((_NW, _NCHUNK, _CH), jnp.float32),
        mesh=_sc_mesh(),
        scratch_types=(
            pltpu.VMEM((_NCHUNK, _CH), jnp.int32),
            pltpu.VMEM((2, _CH), jnp.float32),
            pltpu.SemaphoreType.DMA,
            pltpu.SemaphoreType.DMA,
        ),
    )
    def k(kd_h, invc_h, invn_out, kdv2, invs, semi, semo):
        cid = lax.axis_index("c")
        sid = lax.axis_index("s")
        wid = cid * _NS + sid

        pltpu.sync_copy(kd_h.at[wid], kdv2)
        pltpu.async_copy(invc_h.at[kdv2.at[0]], invs.at[0], semi)

        def chunk(j, carry):
            jm2 = lax.rem(j, 2)

            @pl.when(j >= 1)
            def _():
                pltpu.make_async_copy(invn_out.at[0, 0], invs.at[0],
                                      semo).wait()

            @pl.when(j + 1 < _NCHUNK)
            def _():
                pltpu.async_copy(invc_h.at[kdv2.at[j + 1]],
                                 invs.at[1 - jm2], semi)

            pltpu.make_async_copy(invc_h.at[pl.ds(0, _CH)], invs.at[0],
                                  semi).wait()
            pltpu.async_copy(invs.at[jm2], invn_out.at[wid, j], semo)
            return carry

        lax.fori_loop(0, _NCHUNK, chunk, 0)
        pltpu.make_async_copy(invn_out.at[0, 0], invs.at[0], semo).wait()

    return k(kd3, invc)


def _sc_edge1(xw1, ks3, dst3, invn3):
    @functools.partial(
        pl.kernel,
        out_type=jax.ShapeDtypeStruct((_NC, _NPAD, _H), jnp.float32),
        mesh=_sc_mesh(),
        scratch_types=(
            pltpu.VMEM((3, _CH), jnp.int32),
            pltpu.VMEM((_NCHUNK, _CH), jnp.int32),
            pltpu.VMEM((3, _CH), jnp.float32),
            pltpu.VMEM((3, _CH, _H), jnp.float32),
            pltpu.VMEM_SHARED((_NPAD, _H), jnp.float32),
            pltpu.SemaphoreType.DMA,
            pltpu.SemaphoreType.DMA,
            pltpu.SemaphoreType.DMA,
            pltpu.SemaphoreType.DMA,
        ),
    )
    def k(xw_h, ks_h, dst_h, invn_h, agg_out,
          ksst, dstv2, invs, rows3, agg_s, semr, semn, semk, semsc):
        cid = lax.axis_index("c")
        sid = lax.axis_index("s")
        wid = cid * _NS + sid

        pltpu.sync_copy(dst_h.at[wid], dstv2)
        for p in range(3):
            pltpu.sync_copy(ks_h.at[wid, p], ksst.at[p])
        _spmem_rows_zero_and_sync(rows3.at[0], agg_s, sid, _CH, _H)
        plsc.subcore_barrier()

        def fire(j, buf):
            pltpu.async_copy(xw_h.at[ksst.at[buf]], rows3.at[buf], semr)
            pltpu.async_copy(invn_h.at[wid, j], invs.at[buf], semn)

        fire(0, 0)

        def chunk(j, carry):
            jm3 = lax.rem(j, 3)
            jn3 = lax.rem(j + 1, 3)

            @pl.when(j >= 2)
            def _():
                pltpu.make_async_copy(xw_h.at[pl.ds(0, _CH)],
                                      rows3.at[0], semsc).wait()

            @pl.when(jnp.logical_and(j >= 2, j + 1 < _NCHUNK))
            def _():
                pltpu.make_async_copy(ks_h.at[0, 0], ksst.at[0], semk).wait()

            @pl.when(j + 1 < _NCHUNK)
            def _():
                fire(j + 1, jn3)

            pltpu.make_async_copy(xw_h.at[pl.ds(0, _CH)],
                                  rows3.at[0], semr).wait()
            pltpu.make_async_copy(invn_h.at[0, 0], invs.at[0], semn).wait()

            @pl.when(j + 3 < _NCHUNK)
            def _():
                pltpu.async_copy(ks_h.at[wid, j + 3], ksst.at[jm3], semk)

            def blk(bb, c2):
                iv = invs[jm3, pl.ds(bb * 16, 16)]
                for ii in range(16):
                    i = bb * 16 + ii
                    inv = jnp.full((16,), iv[ii], jnp.float32)
                    for q in range(_H // 16):
                        sl = pl.ds(q * 16, 16)
                        rows3[jm3, i, sl] = rows3[jm3, i, sl] * inv
                return c2

            lax.fori_loop(0, _CH // 16, blk, 0)
            pltpu.async_copy(rows3.at[jm3], agg_s.at[dstv2.at[j]], semsc,
                             add=True)
            return carry

        lax.fori_loop(0, _NCHUNK, chunk, 0)
        for _p in range(2):
            pltpu.make_async_copy(xw_h.at[pl.ds(0, _CH)], rows3.at[0],
                                  semsc).wait()
        plsc.subcore_barrier()
        _spmem_rows_out(rows3.at[0], agg_s, agg_out, cid, sid, _CH)

    return k(xw1, ks3, dst3, invn3)


def _sc_edge2(xw2, ks, dst3, invn3):
    @functools.partial(
        pl.kernel,
        out_type=jax.ShapeDtypeStruct((_NC, _NPAD, _L), jnp.float32),
        mesh=_sc_mesh(),
        compiler_params=pltpu.CompilerParams(use_tc_tiling_on_sc=False),
        scratch_types=(
            pltpu.VMEM((_EPT,), jnp.int32),
            pltpu.VMEM((_NCHUNK, _CH), jnp.int32),
            pltpu.VMEM((_NCHUNK, _CH), jnp.float32),
            pltpu.VMEM((3, _CH, _L), jnp.float32),
            pltpu.VMEM_SHARED((_NPAD, _L), jnp.float32),
            pltpu.SemaphoreType.DMA,
            pltpu.SemaphoreType.DMA,
        ),
    )
    def k(xw_h, ks_h, dst_h, invn_h, agg_out,
          ksv, dstv2, invnv2, rows3, agg_s, semr, semsc):
        cid = lax.axis_index("c")
        sid = lax.axis_index("s")
        wid = cid * _NS + sid
        base = wid * _EPT

        pltpu.sync_copy(ks_h.at[pl.ds(base, _EPT)], ksv)
        pltpu.sync_copy(dst_h.at[wid], dstv2)
        pltpu.sync_copy(invn_h.at[wid], invnv2)
        _spmem_rows_zero_and_sync(rows3.at[0], agg_s, sid, _CH, _L)
        plsc.subcore_barrier()

        def fire(j, buf):
            pltpu.async_copy(xw_h.at[ksv.at[pl.ds(j * _CH, _CH)]],
                             rows3.at[buf], semr)

        fire(0, 0)

        def chunk(j, carry):
            jm3 = lax.rem(j, 3)
            jn3 = lax.rem(j + 1, 3)

            @pl.when(j >= 2)
            def _():
                pltpu.make_async_copy(xw_h.at[pl.ds(0, _CH)],
                                      rows3.at[0], semsc).wait()

            @pl.when(j + 1 < _NCHUNK)
            def _():
                fire(j + 1, jn3)

            pltpu.make_async_copy(xw_h.at[pl.ds(0, _CH)],
                                  rows3.at[0], semr).wait()

            def blk(bb, c2):
                iv = invnv2[j, pl.ds(bb * 16, 16)]
                for ii in range(16):
                    i = bb * 16 + ii
                    inv = jnp.full((16,), iv[ii], jnp.float32)
                    rows3[jm3, i, pl.ds(0, _L)] = (
                        rows3[jm3, i, pl.ds(0, _L)] * inv)
                return c2

            lax.fori_loop(0, _CH // 16, blk, 0)
            pltpu.async_copy(rows3.at[jm3], agg_s.at[dstv2.at[j]], semsc,
                             add=True)
            return carry

        lax.fori_loop(0, _NCHUNK, chunk, 0)
        for _p in range(2):
            pltpu.make_async_copy(xw_h.at[pl.ds(0, _CH)], rows3.at[0],
                                  semsc).wait()
        plsc.subcore_barrier()
        _spmem_rows_out(rows3.at[0], agg_s, agg_out, cid, sid, _CH)

    return k(xw2, ks, dst3, invn3)


def _tc_pre(emb, wc, bc8, w1c, root1, b18):
    def body(emb_ref, wc_ref, bc_ref, w1c_ref, r1_ref, b1_ref,
             x1_ref, xw1_ref, rt1_ref):
        bc = bc_ref[...][0:1, :]
        b1 = b1_ref[...][0:1, :]
        x = jnp.dot(emb_ref[...], wc_ref[...],
                    preferred_element_type=jnp.float32) + bc
        x1_ref[...] = x
        xw1_ref[...] = jnp.dot(x, w1c_ref[...],
                               preferred_element_type=jnp.float32)
        rt1_ref[...] = jnp.dot(x, r1_ref[...],
                               preferred_element_type=jnp.float32) + b1

    return pl.pallas_call(
        body,
        grid=(_NBLK,),
        in_specs=[
            pl.BlockSpec((_BN, _D), lambda i: (i, 0)),
            pl.BlockSpec((_D, _D), lambda i: (0, 0)),
            pl.BlockSpec((8, _D), lambda i: (0, 0)),
            pl.BlockSpec((_D, _R * _H), lambda i: (0, 0)),
            pl.BlockSpec((_D, _H), lambda i: (0, 0)),
            pl.BlockSpec((8, _H), lambda i: (0, 0)),
        ],
        out_specs=[
            pl.BlockSpec((_BN, _D), lambda i: (i, 0)),
            pl.BlockSpec((_BN, _R * _H), lambda i: (i, 0)),
            pl.BlockSpec((_BN, _H), lambda i: (i, 0)),
        ],
        out_shape=[
            jax.ShapeDtypeStruct((_N, _D), jnp.float32),
            jax.ShapeDtypeStruct((_N, _R * _H), jnp.float32),
            jax.ShapeDtypeStruct((_N, _H), jnp.float32),
        ],
    )(emb, wc, bc8, w1c, root1, b18)


def _tc_invc(cnt2):
    def body(cnt_ref, invc_ref):
        c = cnt_ref[0] + cnt_ref[1]
        invc_ref[...] = 1.0 / jnp.maximum(c, 1.0)

    return pl.pallas_call(
        body,
        out_shape=jax.ShapeDtypeStruct((_NR // 128, 128), jnp.float32),
    )(cnt2)


def _tc_mid(agg1, rt1, w2c, root2, b28):
    def body(agg_ref, rt1_ref, w2c_ref, r2_ref, b2_ref, xw2_ref, rt2_ref):
        b2 = b2_ref[...][0:1, :]
        x2 = jax.nn.relu(agg_ref[0] + agg_ref[1] + rt1_ref[...])
        xw2_ref[...] = jnp.dot(x2, w2c_ref[...],
                               preferred_element_type=jnp.float32)
        rt2_ref[...] = jnp.dot(x2, r2_ref[...],
                               preferred_element_type=jnp.float32) + b2

    return pl.pallas_call(
        body,
        grid=(_NBLK,),
        in_specs=[
            pl.BlockSpec((2, _BN, _H), lambda i: (0, i, 0)),
            pl.BlockSpec((_BN, _H), lambda i: (i, 0)),
            pl.BlockSpec((_H, _R * _L), lambda i: (0, 0)),
            pl.BlockSpec((_H, _L), lambda i: (0, 0)),
            pl.BlockSpec((8, _L), lambda i: (0, 0)),
        ],
        out_specs=[
            pl.BlockSpec((_BN, _R * _L), lambda i: (i, 0)),
            pl.BlockSpec((_BN, _L), lambda i: (i, 0)),
        ],
        out_shape=[
            jax.ShapeDtypeStruct((_N, _R * _L), jnp.float32),
            jax.ShapeDtypeStruct((_N, _L), jnp.float32),
        ],
    )(agg1, rt1, w2c, root2, b28)


def _tc_final(agg2, rt2):
    def body(agg_ref, rt2_ref, out_ref):
        y = agg_ref[0] + agg_ref[1] + rt2_ref[...]
        m = jnp.max(y, axis=-1, keepdims=True)
        e = jnp.exp(y - m)
        out_ref[...] = e / jnp.sum(e, axis=-1, keepdims=True)

    return pl.pallas_call(
        body,
        grid=(_NBLK,),
        in_specs=[
            pl.BlockSpec((2, _BN, _L), lambda i: (0, i, 0)),
            pl.BlockSpec((_BN, _L), lambda i: (i, 0)),
        ],
        out_specs=pl.BlockSpec((_BN, _L), lambda i: (i, 0)),
        out_shape=jax.ShapeDtypeStruct((_N, _L), jnp.float32),
    )(agg2, rt2)


def kernel(embedding, edge_index, edge_type, in_proj_w, in_proj_b,
           out_proj_w, out_proj_b, w1, root1, b1, w2, root2, b2):
    emb = embedding[0]
    wv = in_proj_w[2 * _D:3 * _D]
    bv = in_proj_b[2 * _D:3 * _D]
    wc = wv.T @ out_proj_w.T
    bc = bv @ out_proj_w.T + out_proj_b
    bc8 = jnp.broadcast_to(bc[None, :], (8, _D))
    b18 = jnp.broadcast_to(b1[None, :], (8, _H))
    b28 = jnp.broadcast_to(b2[None, :], (8, _L))
    w1c = w1.transpose(1, 0, 2).reshape(_D, _R * _H)
    w2c = w2.transpose(1, 0, 2).reshape(_H, _R * _L)

    src = edge_index[0]
    dst = edge_index[1]
    rel = edge_type
    dst3 = dst.reshape(_NW, _NCHUNK, _CH)

    cnt2, ks3, kd3 = _sc_counts(src, dst, rel)
    ks = ks3.reshape(_E)
    invc = _tc_invc(cnt2.reshape(2, _NR // 128, 128))
    invn3 = _sc_invn(kd3, invc.reshape(_NR))
    _, xw1, rt1 = _tc_pre(emb, wc, bc8, w1c, root1, b18)
    agg1 = _sc_edge1(xw1.reshape(_NR, _H), ks3, dst3, invn3)
    xw2, rt2 = _tc_mid(agg1, rt1, w2c, root2, b28)
    agg2 = _sc_edge2(xw2.reshape(_NR, _L), ks, dst3, invn3)
    return _tc_final(agg2, rt2)
